# baseline (speedup 1.0000x reference)
import functools

import jax
import jax.numpy as jnp
from jax import lax
from jax.experimental import pallas as pl
from jax.experimental.pallas import tpu as pltpu
from jax.experimental.pallas import tpu_sc as plsc


def _tc_body(x_ref, pos_ref, out_ref):
    out_ref[...] = x_ref[...] + pos_ref[...][None, :, :]


def _tc_add(x, pos_embedding):
    B, S, D = x.shape
    BS = 2048
    grid = (S // BS, B)
    return pl.pallas_call(
        _tc_body,
        grid=grid,
        in_specs=[
            pl.BlockSpec((1, BS, D), lambda s, b: (b, s, 0)),
            pl.BlockSpec((BS, D), lambda s, b: (s, 0)),
        ],
        out_specs=pl.BlockSpec((1, BS, D), lambda s, b: (b, s, 0)),
        out_shape=jax.ShapeDtypeStruct((B, S, D), x.dtype),
        compiler_params=pltpu.CompilerParams(
            dimension_semantics=("arbitrary", "arbitrary"),
        ),
    )(x, pos_embedding[:S])


_NC = 2
_NS = 16
_NW = _NC * _NS
_LANES = 16


def _sc_add(xf, posf, B, S, D):
    rows_per_w = S // _NW
    CH = 32
    NE = CH * D
    NCH = rows_per_w // CH
    U = 8

    mesh = plsc.VectorSubcoreMesh(core_axis_name="c", subcore_axis_name="s")

    @functools.partial(
        pl.kernel,
        mesh=mesh,
        out_type=jax.ShapeDtypeStruct((B * S * D,), jnp.float32),
        scratch_types=[
            pltpu.VMEM((NE,), jnp.float32),
            pltpu.VMEM((NE,), jnp.float32),
        ],
    )
    def k(x_hbm, pos_hbm, out_hbm, pos_v, x_v):
        wid = lax.axis_index("s") * _NC + lax.axis_index("c")
        row_base = wid * rows_per_w

        def chunk_body(c, _):
            row0 = row_base + c * CH
            pltpu.sync_copy(pos_hbm.at[pl.ds(row0 * D, NE)], pos_v)

            def batch_body(b, __):
                xoff = (b * S + row0) * D
                pltpu.sync_copy(x_hbm.at[pl.ds(xoff, NE)], x_v)

                def add_body(j, ___):
                    base = j * (_LANES * U)
                    for u in range(U):
                        o = base + u * _LANES
                        x_v[pl.ds(o, _LANES)] = (
                            x_v[pl.ds(o, _LANES)] + pos_v[pl.ds(o, _LANES)]
                        )
                    return 0

                lax.fori_loop(0, NE // (_LANES * U), add_body, 0)
                pltpu.sync_copy(x_v, out_hbm.at[pl.ds(xoff, NE)])
                return 0

            lax.fori_loop(0, B, batch_body, 0)
            return 0

        lax.fori_loop(0, NCH, chunk_body, 0)

    return k(xf, posf)


def kernel(x, pos_embedding):
    B, S, D = x.shape
    xf = x.reshape(B * S * D)
    posf = pos_embedding[:S].reshape(S * D)
    out = _sc_add(xf, posf, B, S, D)
    return out.reshape(B, S, D)


# speedup vs baseline: 5.1553x; 5.1553x over previous
import functools

import jax
import jax.numpy as jnp
from jax import lax
from jax.experimental import pallas as pl
from jax.experimental.pallas import tpu as pltpu
from jax.experimental.pallas import tpu_sc as plsc


def _tc_body(x_ref, pos_ref, out_ref):
    out_ref[...] = x_ref[...] + pos_ref[...][None, :, :]


def _tc_add(x, pos_embedding):
    B, S, D = x.shape
    BS = 2048
    grid = (S // BS, B)
    return pl.pallas_call(
        _tc_body,
        grid=grid,
        in_specs=[
            pl.BlockSpec((1, BS, D), lambda s, b: (b, s, 0)),
            pl.BlockSpec((BS, D), lambda s, b: (s, 0)),
        ],
        out_specs=pl.BlockSpec((1, BS, D), lambda s, b: (b, s, 0)),
        out_shape=jax.ShapeDtypeStruct((B, S, D), x.dtype),
        compiler_params=pltpu.CompilerParams(
            dimension_semantics=("arbitrary", "arbitrary"),
        ),
    )(x, pos_embedding[:S])


_NC = 2
_NS = 16
_NW = _NC * _NS
_LANES = 16


def _sc_add(xf, posf, B, S, D):
    rows_per_w = S // _NW
    CH = 32
    NE = CH * D
    NCH = rows_per_w // CH
    U = 8

    mesh = plsc.VectorSubcoreMesh(core_axis_name="c", subcore_axis_name="s")

    @functools.partial(
        pl.kernel,
        mesh=mesh,
        out_type=jax.ShapeDtypeStruct((B * S * D,), jnp.float32),
        scratch_types=[
            pltpu.VMEM((NE,), jnp.float32),
            pltpu.VMEM((NE,), jnp.float32),
        ],
    )
    def k(x_hbm, pos_hbm, out_hbm, pos_v, x_v):
        wid = lax.axis_index("s") * _NC + lax.axis_index("c")
        row_base = wid * rows_per_w

        def chunk_body(c, _):
            row0 = row_base + c * CH
            pltpu.sync_copy(pos_hbm.at[pl.ds(row0 * D, NE)], pos_v)

            def batch_body(b, __):
                xoff = (b * S + row0) * D
                pltpu.sync_copy(x_hbm.at[pl.ds(xoff, NE)], x_v)

                def add_body(j, ___):
                    base = j * (_LANES * U)
                    for u in range(U):
                        o = base + u * _LANES
                        x_v[pl.ds(o, _LANES)] = (
                            x_v[pl.ds(o, _LANES)] + pos_v[pl.ds(o, _LANES)]
                        )
                    return 0

                lax.fori_loop(0, NE // (_LANES * U), add_body, 0)
                pltpu.sync_copy(x_v, out_hbm.at[pl.ds(xoff, NE)])
                return 0

            lax.fori_loop(0, B, batch_body, 0)
            return 0

        lax.fori_loop(0, NCH, chunk_body, 0)

    return k(xf, posf)


def kernel(x, pos_embedding):
    return _tc_add(x, pos_embedding)
